# baseline (speedup 1.0000x reference)
import functools

import jax
import jax.numpy as jnp
from jax import lax
from jax.experimental import pallas as pl
from jax.experimental.pallas import tpu as pltpu
from jax.experimental.pallas import tpu_sc as plsc

NC = 2
NS = 16
NW = NC * NS
MP = 10240
CH = 80
MS = MP // NS
NB = MS // CH

F32 = jnp.float32
I32 = jnp.int32

_MESH = plsc.VectorSubcoreMesh(
    core_axis_name="c", subcore_axis_name="s", num_cores=NC, num_subcores=NS)


def _fill_iota(iota_v, base_row):
    ramp = lax.iota(I32, 16)
    for t in range(NB):
        for k in range(CH // 16):
            iota_v[t, pl.ds(k * 16, 16)] = ramp + (base_row + t * CH + k * 16)


def _seg_kernel(N):
    SPT = N // NS
    NCH = SPT // CH

    @functools.partial(
        pl.kernel,
        out_type=(
            jax.ShapeDtypeStruct((MP, 128), F32),
            jax.ShapeDtypeStruct((MP, 128), F32),
        ),
        mesh=_MESH,
        scratch_types=[
            pltpu.VMEM((4, CH, 128), F32),
            pltpu.VMEM((25, CH), I32),
            pltpu.VMEM((NB, CH), I32),
            pltpu.VMEM_SHARED((MP, 128), F32),
            pltpu.SemaphoreType.DMA((4,)),
            pltpu.SemaphoreType.DMA((2,)),
            pltpu.SemaphoreType.DMA((2,)),
            pltpu.SemaphoreType.DMA((2,)),
        ],
    )
    def seg_sum(feat_hbm, idxr_hbm, z_hbm, one_hbm,
                osum_hbm, ocnt_hbm,
                f_v, idx_v, iota_v, acc, sem_f, sem_sc, sem_r, sem_o):
        c = lax.axis_index("c")
        s = lax.axis_index("s")
        base_row = s * MS
        base_chunk = s * NCH
        _fill_iota(iota_v, base_row)

        pltpu.sync_copy(z_hbm, f_v.at[0])
        for t in range(NB):
            pltpu.sync_copy(f_v.at[0], acc.at[iota_v.at[t]])
        pltpu.sync_copy(one_hbm, f_v.at[1])
        pltpu.sync_copy(idxr_hbm.at[s, 0], idx_v)
        plsc.subcore_barrier()

        @pl.when(c == 0)
        def _sum_loop():
            for j in range(3):
                pltpu.async_copy(
                    feat_hbm.at[pl.ds((base_chunk + j) * CH, CH), :],
                    f_v.at[j], sem_f.at[j])

            def chunk(k, carry):
                p = lax.rem(k, 4)
                ps = lax.rem(k, 2)
                kb = lax.rem(k, 25)
                b = (base_chunk + k) * CH

                @pl.when(k >= 1)
                def _drain():
                    pltpu.make_async_copy(
                        f_v.at[0], acc.at[idx_v.at[0]],
                        sem_sc.at[1 - ps]).wait()

                @pl.when(jnp.logical_and(kb == 0, k > 0))
                def _reload():
                    pltpu.sync_copy(idxr_hbm.at[s, k // 25], idx_v)

                pltpu.make_async_copy(
                    feat_hbm.at[pl.ds(b, CH), :], f_v.at[p], sem_f.at[p]).wait()
                pltpu.async_copy(
                    f_v.at[p], acc.at[idx_v.at[kb]], sem_sc.at[ps], add=True)

                @pl.when(k + 3 < NCH)
                def _next():
                    p3 = lax.rem(k + 3, 4)
                    pltpu.async_copy(
                        feat_hbm.at[pl.ds(b + 3 * CH, CH), :], f_v.at[p3],
                        sem_f.at[p3])

                return carry

            lax.fori_loop(0, NCH, chunk, 0)
            pltpu.make_async_copy(
                f_v.at[0], acc.at[idx_v.at[0]],
                sem_sc.at[(NCH - 1) % 2]).wait()

        @pl.when(c == 1)
        def _cnt_loop():
            def chunk(k, carry):
                p = lax.rem(k, 2)
                kb = lax.rem(k, 25)

                @pl.when(k >= 1)
                def _drain():
                    pltpu.make_async_copy(
                        f_v.at[1], acc.at[idx_v.at[0]], sem_sc.at[1 - p]).wait()

                @pl.when(jnp.logical_and(kb == 0, k > 0))
                def _reload():
                    pltpu.sync_copy(idxr_hbm.at[s, k // 25], idx_v)

                pltpu.async_copy(
                    f_v.at[1], acc.at[idx_v.at[kb]], sem_sc.at[p], add=True)
                return carry

            lax.fori_loop(0, NCH, chunk, 0)
            pltpu.make_async_copy(
                f_v.at[1], acc.at[idx_v.at[0]],
                sem_sc.at[(NCH - 1) % 2]).wait()

        plsc.subcore_barrier()

        for ci, oref_name in enumerate(("sum", "cnt")):
            oref = osum_hbm if ci == 0 else ocnt_hbm

            @pl.when(c == ci)
            def _writeout(oref=oref):
                pltpu.async_copy(acc.at[iota_v.at[0]], f_v.at[0], sem_r.at[0])
                for t in range(NB):
                    pltpu.make_async_copy(
                        acc.at[iota_v.at[t]], f_v.at[t % 2],
                        sem_r.at[t % 2]).wait()
                    if t + 1 < NB:
                        if t >= 1:
                            pltpu.make_async_copy(
                                f_v.at[(t + 1) % 2],
                                oref.at[pl.ds(base_row + (t - 1) * CH, CH), :],
                                sem_o.at[(t + 1) % 2]).wait()
                        pltpu.async_copy(acc.at[iota_v.at[t + 1]],
                                         f_v.at[(t + 1) % 2],
                                         sem_r.at[(t + 1) % 2])
                    pltpu.async_copy(
                        f_v.at[t % 2],
                        oref.at[pl.ds(base_row + t * CH, CH), :],
                        sem_o.at[t % 2])
                for j in range(2):
                    tt = NB - 2 + j
                    pltpu.make_async_copy(
                        f_v.at[tt % 2],
                        oref.at[pl.ds(base_row + tt * CH, CH), :],
                        sem_o.at[tt % 2]).wait()

    return seg_sum


def _concat_body(rgb_ref, xyz_ref, out_ref):
    a = rgb_ref[...]
    b = xyz_ref[...]
    out_ref[...] = jnp.concatenate(
        [a[:, :64], b[:, :64], a[:, 64:], b[:, 64:]], axis=1)


def _weights_body(sum_ref, cnt_ref, w_ref, out_ref):
    cnt = cnt_ref[:, 0:1]
    avg = sum_ref[...] / jnp.maximum(cnt, 1.0)
    z = lax.dot_general(avg, w_ref[...], (((1,), (1,)), ((), ())),
                        preferred_element_type=F32)
    out_ref[...] = 1.0 / (1.0 + jnp.exp(-z))


def _gather_mul_kernel(N):
    SPT = N // NW
    NCH = SPT // CH

    @functools.partial(
        pl.kernel,
        out_type=jax.ShapeDtypeStruct((N, 128), F32),
        mesh=_MESH,
        scratch_types=[
            pltpu.VMEM((25, CH), I32),
            pltpu.VMEM((2, CH, 128), F32),
            pltpu.VMEM((2, CH, 128), F32),
            pltpu.VMEM((NB, CH), I32),
            pltpu.VMEM_SHARED((MP, 128), F32),
            pltpu.SemaphoreType.DMA((2,)),
            pltpu.SemaphoreType.DMA((2,)),
            pltpu.SemaphoreType.DMA((2,)),
            pltpu.SemaphoreType.DMA,
        ],
    )
    def gather_mul(feat_hbm, idxr_hbm, w_hbm, out_hbm,
                   idx_v, w_v, f_v, iota_v, spw,
                   sem_g, sem_f, sem_s, sem_w):
        c = lax.axis_index("c")
        s = lax.axis_index("s")
        wid = c * NS + s
        base_chunk = wid * NCH
        base_row = s * MS
        _fill_iota(iota_v, base_row)

        for t in range(NB):
            pltpu.async_copy(
                w_hbm.at[pl.ds(base_row + t * CH, CH), :], w_v.at[0],
                sem_w).wait()
            pltpu.sync_copy(w_v.at[0], spw.at[iota_v.at[t]])
        pltpu.sync_copy(idxr_hbm.at[wid, 0], idx_v)
        plsc.subcore_barrier()

        pltpu.async_copy(spw.at[idx_v.at[0]], w_v.at[0], sem_g.at[0])
        pltpu.async_copy(
            feat_hbm.at[pl.ds(base_chunk * CH, CH), :], f_v.at[0], sem_f.at[0])

        def chunk(k, carry):
            p = lax.rem(k, 2)
            q = 1 - p
            b = (base_chunk + k) * CH

            pltpu.make_async_copy(
                spw.at[idx_v.at[0]], w_v.at[p], sem_g.at[p]).wait()
            pltpu.make_async_copy(
                feat_hbm.at[pl.ds(b, CH), :], f_v.at[p], sem_f.at[p]).wait()

            @pl.when(jnp.logical_and(lax.rem(k + 1, 25) == 0, k + 1 < NCH))
            def _reload():
                pltpu.sync_copy(idxr_hbm.at[wid, (k + 1) // 25], idx_v)

            @pl.when(k + 1 < NCH)
            def _next():
                @pl.when(k >= 1)
                def _drain():
                    pltpu.make_async_copy(
                        w_v.at[q], out_hbm.at[pl.ds(b - CH, CH), :],
                        sem_s.at[q]).wait()

                pltpu.async_copy(spw.at[idx_v.at[lax.rem(k + 1, 25)]],
                                 w_v.at[q], sem_g.at[q])
                pltpu.async_copy(feat_hbm.at[pl.ds(b + CH, CH), :], f_v.at[q],
                                 sem_f.at[q])

            @plsc.parallel_loop(0, CH, 1, unroll=8)
            def row(r):
                for j in range(8):
                    w_v[p, r, pl.ds(j * 16, 16)] = (
                        w_v[p, r, pl.ds(j * 16, 16)]
                        * f_v[p, r, pl.ds(j * 16, 16)]
                    )

            pltpu.async_copy(w_v.at[p], out_hbm.at[pl.ds(b, CH), :],
                             sem_s.at[p])
            return carry

        lax.fori_loop(0, NCH, chunk, 0)
        for j in range(2):
            kk = NCH - 2 + j
            pltpu.make_async_copy(
                w_v.at[kk % 2],
                out_hbm.at[pl.ds((base_chunk + kk) * CH, CH), :],
                sem_s.at[kk % 2]).wait()

    return gather_mul


def kernel(intersect_rgb_feat, intersect_voxel_feat, miss_ray_intersect_idx,
           total_miss_sample_num, W):
    N = intersect_rgb_feat.shape[0]
    idxa = miss_ray_intersect_idx.reshape(NS, 10, (N // NS) // CH // 10, CH)
    idxr = miss_ray_intersect_idx.reshape(NW, 5, (N // NW) // CH // 5, CH)
    z = jnp.zeros((CH, 128), F32)
    on1 = jnp.ones((CH, 128), F32)

    feat = jnp.concatenate([intersect_rgb_feat, intersect_voxel_feat],
                           axis=-1)

    sums, cnts = _seg_kernel(N)(feat, idxa, z, on1)

    BM = 1024
    weights = pl.pallas_call(
        _weights_body,
        grid=(MP // BM,),
        in_specs=[
            pl.BlockSpec((BM, 128), lambda i: (i, 0)),
            pl.BlockSpec((BM, 128), lambda i: (i, 0)),
            pl.BlockSpec((128, 128), lambda i: (0, 0)),
        ],
        out_specs=pl.BlockSpec((BM, 128), lambda i: (i, 0)),
        out_shape=jax.ShapeDtypeStruct((MP, 128), F32),
    )(sums, cnts, W)

    out = _gather_mul_kernel(N)(feat, idxr, weights)
    return out


# speedup vs baseline: 1.1337x; 1.1337x over previous
import functools

import jax
import jax.numpy as jnp
from jax import lax
from jax.experimental import pallas as pl
from jax.experimental.pallas import tpu as pltpu
from jax.experimental.pallas import tpu_sc as plsc

NC = 2
NS = 16
NW = NC * NS
MP = 10240
CH = 80
MS = MP // NS
NB = MS // CH

F32 = jnp.float32
I32 = jnp.int32

_MESH = plsc.VectorSubcoreMesh(
    core_axis_name="c", subcore_axis_name="s", num_cores=NC, num_subcores=NS)


def _fill_iota(iota_v, base_row):
    ramp = lax.iota(I32, 16)
    for t in range(NB):
        for k in range(CH // 16):
            iota_v[t, pl.ds(k * 16, 16)] = ramp + (base_row + t * CH + k * 16)


def _cnt_kernel(N):
    SPT = N // NW
    NCH = SPT // CH

    @functools.partial(
        pl.kernel,
        out_type=jax.ShapeDtypeStruct((NC, MP, 128), F32),
        mesh=_MESH,
        scratch_types=[
            pltpu.VMEM((2, CH, 128), F32),
            pltpu.VMEM((25, CH), I32),
            pltpu.VMEM((NB, CH), I32),
            pltpu.VMEM_SHARED((MP, 128), F32),
            pltpu.SemaphoreType.DMA((2,)),
            pltpu.SemaphoreType.DMA((2,)),
            pltpu.SemaphoreType.DMA((2,)),
        ],
    )
    def cnt_sum(idxr_hbm, z_hbm, one_hbm, ocnt_hbm,
                aux2_v, idx_v, iota_v, acc, sem_sc, sem_r, sem_o):
        aux_v = aux2_v.at[0]
        c = lax.axis_index("c")
        s = lax.axis_index("s")
        wid = c * NS + s
        base_row = s * MS
        _fill_iota(iota_v, base_row)

        pltpu.sync_copy(z_hbm, aux_v)
        for t in range(NB):
            pltpu.sync_copy(aux_v, acc.at[iota_v.at[t]])
        pltpu.sync_copy(one_hbm, aux_v)
        pltpu.sync_copy(idxr_hbm.at[wid, 0], idx_v)
        plsc.subcore_barrier()

        def chunk(k, carry):
            p = lax.rem(k, 2)
            kb = lax.rem(k, 25)

            @pl.when(k >= 1)
            def _drain():
                pltpu.make_async_copy(
                    aux_v, acc.at[idx_v.at[0]], sem_sc.at[1 - p]).wait()

            @pl.when(jnp.logical_and(kb == 0, k > 0))
            def _reload():
                pltpu.sync_copy(idxr_hbm.at[wid, k // 25], idx_v)

            pltpu.async_copy(
                aux_v, acc.at[idx_v.at[kb]], sem_sc.at[p], add=True)
            return carry

        lax.fori_loop(0, NCH, chunk, 0)
        pltpu.make_async_copy(
            aux_v, acc.at[idx_v.at[0]], sem_sc.at[(NCH - 1) % 2]).wait()
        plsc.subcore_barrier()

        pltpu.async_copy(acc.at[iota_v.at[0]], aux2_v.at[0], sem_r.at[0])
        for t in range(NB):
            pltpu.make_async_copy(
                acc.at[iota_v.at[t]], aux2_v.at[t % 2], sem_r.at[t % 2]).wait()
            if t + 1 < NB:
                if t >= 1:
                    pltpu.make_async_copy(
                        aux2_v.at[(t + 1) % 2],
                        ocnt_hbm.at[c, pl.ds(base_row + (t - 1) * CH, CH), :],
                        sem_o.at[(t + 1) % 2]).wait()
                pltpu.async_copy(acc.at[iota_v.at[t + 1]],
                                 aux2_v.at[(t + 1) % 2], sem_r.at[(t + 1) % 2])
            pltpu.async_copy(
                aux2_v.at[t % 2],
                ocnt_hbm.at[c, pl.ds(base_row + t * CH, CH), :],
                sem_o.at[t % 2])
        for j in range(2):
            tt = NB - 2 + j
            pltpu.make_async_copy(
                aux2_v.at[tt % 2],
                ocnt_hbm.at[c, pl.ds(base_row + tt * CH, CH), :],
                sem_o.at[tt % 2]).wait()

    return cnt_sum


def _sum_kernel(N):
    SPT = N // NW
    NCH = SPT // CH

    @functools.partial(
        pl.kernel,
        out_type=jax.ShapeDtypeStruct((NC, MP, 128), F32),
        mesh=_MESH,
        scratch_types=[
            pltpu.VMEM((4, CH, 128), F32),
            pltpu.VMEM((25, CH), I32),
            pltpu.VMEM((NB, CH), I32),
            pltpu.VMEM_SHARED((MP, 128), F32),
            pltpu.SemaphoreType.DMA((4,)),
            pltpu.SemaphoreType.DMA((2,)),
            pltpu.SemaphoreType.DMA((2,)),
            pltpu.SemaphoreType.DMA((2,)),
        ],
    )
    def seg_sum(feat_hbm, idxr_hbm, z_hbm, osum_hbm,
                f_v, idx_v, iota_v, acc, sem_f, sem_sc, sem_r, sem_o):
        c = lax.axis_index("c")
        s = lax.axis_index("s")
        wid = c * NS + s
        base_row = s * MS
        base_chunk = wid * NCH
        _fill_iota(iota_v, base_row)

        pltpu.sync_copy(z_hbm, f_v.at[0])
        for t in range(NB):
            pltpu.sync_copy(f_v.at[0], acc.at[iota_v.at[t]])
        pltpu.sync_copy(idxr_hbm.at[wid, 0], idx_v)
        plsc.subcore_barrier()

        for j in range(3):
            pltpu.async_copy(
                feat_hbm.at[pl.ds((base_chunk + j) * CH, CH), :],
                f_v.at[j], sem_f.at[j])

        def chunk(k, carry):
            p = lax.rem(k, 4)
            ps = lax.rem(k, 2)
            kb = lax.rem(k, 25)
            b = (base_chunk + k) * CH

            @pl.when(k >= 1)
            def _drain():
                pltpu.make_async_copy(
                    f_v.at[0], acc.at[idx_v.at[0]], sem_sc.at[1 - ps]).wait()

            @pl.when(jnp.logical_and(kb == 0, k > 0))
            def _reload():
                pltpu.sync_copy(idxr_hbm.at[wid, k // 25], idx_v)

            pltpu.make_async_copy(
                feat_hbm.at[pl.ds(b, CH), :], f_v.at[p], sem_f.at[p]).wait()
            pltpu.async_copy(
                f_v.at[p], acc.at[idx_v.at[kb]], sem_sc.at[ps], add=True)

            @pl.when(k + 3 < NCH)
            def _next():
                p3 = lax.rem(k + 3, 4)
                pltpu.async_copy(
                    feat_hbm.at[pl.ds(b + 3 * CH, CH), :], f_v.at[p3],
                    sem_f.at[p3])

            return carry

        lax.fori_loop(0, NCH, chunk, 0)
        pltpu.make_async_copy(
            f_v.at[0], acc.at[idx_v.at[0]], sem_sc.at[(NCH - 1) % 2]).wait()
        plsc.subcore_barrier()

        pltpu.async_copy(acc.at[iota_v.at[0]], f_v.at[0], sem_r.at[0])
        for t in range(NB):
            pltpu.make_async_copy(
                acc.at[iota_v.at[t]], f_v.at[t % 2], sem_r.at[t % 2]).wait()
            if t + 1 < NB:
                if t >= 1:
                    pltpu.make_async_copy(
                        f_v.at[(t + 1) % 2],
                        osum_hbm.at[c, pl.ds(base_row + (t - 1) * CH, CH), :],
                        sem_o.at[(t + 1) % 2]).wait()
                pltpu.async_copy(acc.at[iota_v.at[t + 1]],
                                 f_v.at[(t + 1) % 2], sem_r.at[(t + 1) % 2])
            pltpu.async_copy(
                f_v.at[t % 2],
                osum_hbm.at[c, pl.ds(base_row + t * CH, CH), :],
                sem_o.at[t % 2])
        for j in range(2):
            tt = NB - 2 + j
            pltpu.make_async_copy(
                f_v.at[tt % 2],
                osum_hbm.at[c, pl.ds(base_row + tt * CH, CH), :],
                sem_o.at[tt % 2]).wait()

    return seg_sum


def _concat_body(rgb_ref, xyz_ref, out_ref):
    a = rgb_ref[...]
    b = xyz_ref[...]
    out_ref[...] = jnp.concatenate(
        [a[:, :64], b[:, :64], a[:, 64:], b[:, 64:]], axis=1)


def _weights_body(sum_ref, cnt_ref, w_ref, out_ref):
    ssum = sum_ref[0] + sum_ref[1]
    cnt = cnt_ref[0][:, 0:1] + cnt_ref[1][:, 0:1]
    avg = ssum / jnp.maximum(cnt, 1.0)
    z = lax.dot_general(avg, w_ref[...], (((1,), (1,)), ((), ())),
                        preferred_element_type=F32)
    out_ref[...] = 1.0 / (1.0 + jnp.exp(-z))


def _gather_mul_kernel(N):
    SPT = N // NW
    NCH = SPT // CH

    @functools.partial(
        pl.kernel,
        out_type=jax.ShapeDtypeStruct((N, 128), F32),
        mesh=_MESH,
        scratch_types=[
            pltpu.VMEM((25, CH), I32),
            pltpu.VMEM((2, CH, 128), F32),
            pltpu.VMEM((2, CH, 128), F32),
            pltpu.VMEM((NB, CH), I32),
            pltpu.VMEM_SHARED((MP, 128), F32),
            pltpu.SemaphoreType.DMA((2,)),
            pltpu.SemaphoreType.DMA((2,)),
            pltpu.SemaphoreType.DMA((2,)),
            pltpu.SemaphoreType.DMA,
        ],
    )
    def gather_mul(feat_hbm, idxr_hbm, w_hbm, out_hbm,
                   idx_v, w_v, f_v, iota_v, spw,
                   sem_g, sem_f, sem_s, sem_w):
        c = lax.axis_index("c")
        s = lax.axis_index("s")
        wid = c * NS + s
        base_chunk = wid * NCH
        base_row = s * MS
        _fill_iota(iota_v, base_row)

        for t in range(NB):
            pltpu.async_copy(
                w_hbm.at[pl.ds(base_row + t * CH, CH), :], w_v.at[0],
                sem_w).wait()
            pltpu.sync_copy(w_v.at[0], spw.at[iota_v.at[t]])
        pltpu.sync_copy(idxr_hbm.at[wid, 0], idx_v)
        plsc.subcore_barrier()

        pltpu.async_copy(spw.at[idx_v.at[0]], w_v.at[0], sem_g.at[0])
        pltpu.async_copy(
            feat_hbm.at[pl.ds(base_chunk * CH, CH), :], f_v.at[0], sem_f.at[0])

        def chunk(k, carry):
            p = lax.rem(k, 2)
            q = 1 - p
            b = (base_chunk + k) * CH

            pltpu.make_async_copy(
                spw.at[idx_v.at[0]], w_v.at[p], sem_g.at[p]).wait()
            pltpu.make_async_copy(
                feat_hbm.at[pl.ds(b, CH), :], f_v.at[p], sem_f.at[p]).wait()

            @pl.when(jnp.logical_and(lax.rem(k + 1, 25) == 0, k + 1 < NCH))
            def _reload():
                pltpu.sync_copy(idxr_hbm.at[wid, (k + 1) // 25], idx_v)

            @pl.when(k + 1 < NCH)
            def _next():
                @pl.when(k >= 1)
                def _drain():
                    pltpu.make_async_copy(
                        w_v.at[q], out_hbm.at[pl.ds(b - CH, CH), :],
                        sem_s.at[q]).wait()

                pltpu.async_copy(spw.at[idx_v.at[lax.rem(k + 1, 25)]],
                                 w_v.at[q], sem_g.at[q])
                pltpu.async_copy(feat_hbm.at[pl.ds(b + CH, CH), :], f_v.at[q],
                                 sem_f.at[q])

            @plsc.parallel_loop(0, CH, 1, unroll=8)
            def row(r):
                for j in range(8):
                    w_v[p, r, pl.ds(j * 16, 16)] = (
                        w_v[p, r, pl.ds(j * 16, 16)]
                        * f_v[p, r, pl.ds(j * 16, 16)]
                    )

            pltpu.async_copy(w_v.at[p], out_hbm.at[pl.ds(b, CH), :],
                             sem_s.at[p])
            return carry

        lax.fori_loop(0, NCH, chunk, 0)
        for j in range(2):
            kk = NCH - 2 + j
            pltpu.make_async_copy(
                w_v.at[kk % 2],
                out_hbm.at[pl.ds((base_chunk + kk) * CH, CH), :],
                sem_s.at[kk % 2]).wait()

    return gather_mul


def kernel(intersect_rgb_feat, intersect_voxel_feat, miss_ray_intersect_idx,
           total_miss_sample_num, W):
    N = intersect_rgb_feat.shape[0]
    idxr = miss_ray_intersect_idx.reshape(NW, 5, (N // NW) // CH // 5, CH)
    z = jnp.zeros((CH, 128), F32)
    on1 = jnp.ones((CH, 128), F32)

    feat = jnp.concatenate([intersect_rgb_feat, intersect_voxel_feat],
                           axis=-1)

    cnts = _cnt_kernel(N)(idxr, z, on1)
    sums = _sum_kernel(N)(feat, idxr, z)

    BM = 1024
    weights = pl.pallas_call(
        _weights_body,
        grid=(MP // BM,),
        in_specs=[
            pl.BlockSpec((NC, BM, 128), lambda i: (0, i, 0)),
            pl.BlockSpec((NC, BM, 128), lambda i: (0, i, 0)),
            pl.BlockSpec((128, 128), lambda i: (0, 0)),
        ],
        out_specs=pl.BlockSpec((BM, 128), lambda i: (i, 0)),
        out_shape=jax.ShapeDtypeStruct((MP, 128), F32),
    )(sums, cnts, W)

    out = _gather_mul_kernel(N)(feat, idxr, weights)
    return out
